# baseline (device time: 26885 ns/iter reference)
import jax
import jax.numpy as jnp
from jax import lax
from jax.experimental import pallas as pl
from jax.experimental.pallas import tpu as pltpu

N_DEV = 16
M = 1024
K = 512
N = 512

STREAMS = ((0, 768, (1, 4)), (768, 256, (4, 1)))
N_EXCH = 60
RS_ROWS = 3 * (192 + 48 + 64 + 16)


def kernel(t, W):
    def body(t_ref, w_ref, out_ref, stage_ref, comm_ref, ag_ref,
             send_sems, recv_sems):
        i = lax.axis_index("i")

        def group(u):
            g = lax.rem(lax.div(i, u), 4)
            return g, i - g * u

        def peer_at(u, g, gbase, d):
            return gbase + lax.rem(g + d, 4) * u

        barrier_sem = pltpu.get_barrier_semaphore()
        for u in (1, 4):
            g, gbase = group(u)
            for d in (1, 2, 3):
                pl.semaphore_signal(
                    barrier_sem, inc=1,
                    device_id=(peer_at(u, g, gbase, d),),
                    device_id_type=pl.DeviceIdType.MESH,
                )

        def mm(row_lo, rows):
            out_ref[pl.ds(row_lo, rows), :] = jnp.dot(
                t_ref[pl.ds(row_lo, rows), :], w_ref[...],
                preferred_element_type=jnp.float32,
            )

        ctr = {"sem": 0, "comm": 0, "stage": 0}

        stx = []
        for s, (base, R, units) in enumerate(STREAMS):
            H0, H1 = R // 4, R // 16
            g0, gbase0 = group(units[0])
            g1, gbase1 = group(units[1])
            keep_lo = base + g0 * H0
            c = {
                "base": base, "R": R, "units": units, "H0": H0,
                "H1": H1, "g0": g0, "gbase0": gbase0, "g1": g1,
                "gbase1": gbase1, "keep_lo": keep_lo,
                "sem0": ctr["sem"], "comm0": ctr["comm"],
                "stage0": ctr["stage"],
            }
            ctr["sem"] += 12
            ctr["comm"] += 3 * H0
            ctr["stage"] += 3 * H0
            c["sem1"] = ctr["sem"]
            ctr["sem"] += 3
            c["comm1"] = ctr["comm"]
            ctr["comm"] += 3 * H1
            c["stage1"] = ctr["stage"]
            ctr["stage"] += 3 * H1
            stx.append(c)

        for s, c in enumerate(stx):
            for d in (1, 2, 3):
                jm = lax.rem(c["g0"] + d, 4)
                q_lo = c["base"] + jm * c["H0"]
                stage_ref[
                    pl.ds(c["stage0"] + (d - 1) * c["H0"], c["H0"]), :
                ] = jnp.dot(
                    t_ref[pl.ds(q_lo, c["H0"]), :], w_ref[...],
                    preferred_element_type=jnp.float32,
                ).astype(jnp.bfloat16)

        pl.semaphore_wait(barrier_sem, 6)

        rs0 = [[None] * 12 for _ in range(2)]
        for k in (1, 2, 3, 0):
            for s, c in enumerate(stx):
                j = lax.rem(c["g1"] + k, 4)
                off = j * c["H1"]
                for d in (1, 2, 3):
                    r = 4 - d
                    sem = c["sem0"] + (r - 1) * 4 + k
                    rdma = pltpu.make_async_remote_copy(
                        src_ref=stage_ref.at[
                            pl.ds(c["stage0"] + (d - 1) * c["H0"] + off,
                                  c["H1"]), :
                        ],
                        dst_ref=comm_ref.at[
                            pl.ds(c["comm0"] + (r - 1) * c["H0"] + off,
                                  c["H1"]), :
                        ],
                        send_sem=send_sems.at[sem],
                        recv_sem=recv_sems.at[sem],
                        device_id=(peer_at(c["units"][0], c["g0"],
                                           c["gbase0"], d),),
                        device_id_type=pl.DeviceIdType.MESH,
                    )
                    rdma.start()
                    rs0[s][(r - 1) * 4 + k] = rdma

        for s, c in enumerate(stx):
            mm(c["keep_lo"], c["H0"])

        rs1 = [[None] * 3 for _ in range(2)]
        for k in (1, 2, 3, 0):
            for s, c in enumerate(stx):
                H0, H1 = c["H0"], c["H1"]
                j = lax.rem(c["g1"] + k, 4)
                off = j * H1
                row = c["keep_lo"] + off
                for r in (1, 2, 3):
                    rs0[s][(r - 1) * 4 + k].wait()
                out_ref[pl.ds(row, H1), :] += (
                    comm_ref[pl.ds(c["comm0"] + off, H1), :].astype(
                        jnp.float32)
                    + comm_ref[pl.ds(c["comm0"] + H0 + off, H1), :].astype(
                        jnp.float32)
                    + comm_ref[
                        pl.ds(c["comm0"] + 2 * H0 + off, H1), :
                    ].astype(jnp.float32)
                )
                if k == 0:
                    continue
                r = 4 - k
                sem = c["sem1"] + r - 1
                st_off = c["stage1"] + (k - 1) * H1
                stage_ref[pl.ds(st_off, H1), :] = out_ref[
                    pl.ds(row, H1), :
                ].astype(jnp.bfloat16)
                rdma = pltpu.make_async_remote_copy(
                    src_ref=stage_ref.at[pl.ds(st_off, H1), :],
                    dst_ref=comm_ref.at[
                        pl.ds(c["comm1"] + (r - 1) * H1, H1), :
                    ],
                    send_sem=send_sems.at[sem],
                    recv_sem=recv_sems.at[sem],
                    device_id=(peer_at(c["units"][1], c["g1"],
                                       c["gbase1"], k),),
                    device_id_type=pl.DeviceIdType.MESH,
                )
                rdma.start()
                rs1[s][k - 1] = rdma

        def ag_send(c, src_lo, H, level, sem_base, d):
            u = c["units"][level]
            g = c["g0"] if level == 0 else c["g1"]
            gb = c["gbase0"] if level == 0 else c["gbase1"]
            r = 4 - d
            rdma = pltpu.make_async_remote_copy(
                src_ref=ag_ref.at[pl.ds(src_lo, H), :],
                dst_ref=ag_ref.at[pl.ds(src_lo, H), :],
                send_sem=send_sems.at[sem_base + r - 1],
                recv_sem=recv_sems.at[sem_base + r - 1],
                device_id=(peer_at(u, g, gb, d),),
                device_id_type=pl.DeviceIdType.MESH,
            )
            rdma.start()
            return rdma

        ag_ctx = [None, None]
        waiters = [[], []]
        for s, c in enumerate(stx):
            H1 = c["H1"]
            chunk_lo = c["keep_lo"] + c["g1"] * H1
            for rdma in rs1[s]:
                rdma.wait()
            out_ref[pl.ds(chunk_lo, H1), :] += (
                comm_ref[pl.ds(c["comm1"], H1), :].astype(jnp.float32)
                + comm_ref[pl.ds(c["comm1"] + H1, H1), :].astype(
                    jnp.float32)
                + comm_ref[pl.ds(c["comm1"] + 2 * H1, H1), :].astype(
                    jnp.float32)
            )
            ag_ref[pl.ds(chunk_lo, H1), :] = out_ref[
                pl.ds(chunk_lo, H1), :
            ].astype(jnp.bfloat16)
            sem_base = ctr["sem"]
            ctr["sem"] += 3
            a2a = [ag_send(c, chunk_lo, H1, 1, sem_base, d)
                   for d in (1, 2, 3)]
            sem_base = ctr["sem"]
            ctr["sem"] += 3
            waiters[s].extend(
                ag_send(c, chunk_lo, H1, 0, sem_base, d)
                for d in (1, 2, 3)
            )
            ag_ctx[s] = (a2a, c["keep_lo"])

        for d in (1, 2, 3):
            for s in (1, 0):
                c = stx[s]
                a2a, block2_lo = ag_ctx[s]
                a2a[d - 1].wait()
                piece_lo = block2_lo + lax.rem(c["g1"] + 4 - d, 4) * c["H1"]
                sem_base = ctr["sem"]
                ctr["sem"] += 3
                waiters[s].extend(
                    ag_send(c, piece_lo, c["H1"], 0, sem_base, dd)
                    for dd in (1, 2, 3)
                )

        for s in (1, 0):
            for rdma in waiters[s]:
                rdma.wait()
            base, R, _ = STREAMS[s]
            out_ref[pl.ds(base, R), :] = ag_ref[
                pl.ds(base, R), :
            ].astype(jnp.float32)

    return pl.pallas_call(
        body,
        out_shape=jax.ShapeDtypeStruct((M, N), jnp.float32),
        in_specs=[
            pl.BlockSpec(memory_space=pltpu.VMEM),
            pl.BlockSpec(memory_space=pltpu.VMEM),
        ],
        out_specs=pl.BlockSpec(memory_space=pltpu.VMEM),
        scratch_shapes=[
            pltpu.VMEM((RS_ROWS, N), jnp.bfloat16),
            pltpu.VMEM((RS_ROWS, N), jnp.bfloat16),
            pltpu.VMEM((M, N), jnp.bfloat16),
            pltpu.SemaphoreType.DMA((N_EXCH,)),
            pltpu.SemaphoreType.DMA((N_EXCH,)),
        ],
        compiler_params=pltpu.CompilerParams(collective_id=0),
    )(t, W)


# device time: 26238 ns/iter; 1.0247x vs baseline; 1.0247x over previous
import jax
import jax.numpy as jnp
from jax import lax
from jax.experimental import pallas as pl
from jax.experimental.pallas import tpu as pltpu

N_DEV = 16
M = 1024
K = 512
N = 512

STREAMS = ((0, 640, (1, 4)), (640, 384, (4, 1)))
N_EXCH = 60
RS_ROWS = 3 * (160 + 40 + 96 + 24)


def kernel(t, W):
    def body(t_ref, w_ref, out_ref, stage_ref, comm_ref, ag_ref,
             send_sems, recv_sems):
        i = lax.axis_index("i")

        def group(u):
            g = lax.rem(lax.div(i, u), 4)
            return g, i - g * u

        def peer_at(u, g, gbase, d):
            return gbase + lax.rem(g + d, 4) * u

        barrier_sem = pltpu.get_barrier_semaphore()
        for u in (1, 4):
            g, gbase = group(u)
            for d in (1, 2, 3):
                pl.semaphore_signal(
                    barrier_sem, inc=1,
                    device_id=(peer_at(u, g, gbase, d),),
                    device_id_type=pl.DeviceIdType.MESH,
                )

        def mm(row_lo, rows):
            out_ref[pl.ds(row_lo, rows), :] = jnp.dot(
                t_ref[pl.ds(row_lo, rows), :], w_ref[...],
                preferred_element_type=jnp.float32,
            )

        ctr = {"sem": 0, "comm": 0, "stage": 0}

        stx = []
        for s, (base, R, units) in enumerate(STREAMS):
            H0, H1 = R // 4, R // 16
            g0, gbase0 = group(units[0])
            g1, gbase1 = group(units[1])
            keep_lo = base + g0 * H0
            c = {
                "base": base, "R": R, "units": units, "H0": H0,
                "H1": H1, "g0": g0, "gbase0": gbase0, "g1": g1,
                "gbase1": gbase1, "keep_lo": keep_lo,
                "sem0": ctr["sem"], "comm0": ctr["comm"],
                "stage0": ctr["stage"],
            }
            ctr["sem"] += 12
            ctr["comm"] += 3 * H0
            ctr["stage"] += 3 * H0
            c["sem1"] = ctr["sem"]
            ctr["sem"] += 3
            c["comm1"] = ctr["comm"]
            ctr["comm"] += 3 * H1
            c["stage1"] = ctr["stage"]
            ctr["stage"] += 3 * H1
            stx.append(c)

        for s, c in enumerate(stx):
            for d in (1, 2, 3):
                jm = lax.rem(c["g0"] + d, 4)
                q_lo = c["base"] + jm * c["H0"]
                stage_ref[
                    pl.ds(c["stage0"] + (d - 1) * c["H0"], c["H0"]), :
                ] = jnp.dot(
                    t_ref[pl.ds(q_lo, c["H0"]), :], w_ref[...],
                    preferred_element_type=jnp.float32,
                ).astype(jnp.bfloat16)

        pl.semaphore_wait(barrier_sem, 6)

        rs0 = [[None] * 12 for _ in range(2)]
        for k in (1, 2, 3, 0):
            for s, c in enumerate(stx):
                j = lax.rem(c["g1"] + k, 4)
                off = j * c["H1"]
                for d in (1, 2, 3):
                    r = 4 - d
                    sem = c["sem0"] + (r - 1) * 4 + k
                    rdma = pltpu.make_async_remote_copy(
                        src_ref=stage_ref.at[
                            pl.ds(c["stage0"] + (d - 1) * c["H0"] + off,
                                  c["H1"]), :
                        ],
                        dst_ref=comm_ref.at[
                            pl.ds(c["comm0"] + (r - 1) * c["H0"] + off,
                                  c["H1"]), :
                        ],
                        send_sem=send_sems.at[sem],
                        recv_sem=recv_sems.at[sem],
                        device_id=(peer_at(c["units"][0], c["g0"],
                                           c["gbase0"], d),),
                        device_id_type=pl.DeviceIdType.MESH,
                    )
                    rdma.start()
                    rs0[s][(r - 1) * 4 + k] = rdma

        for s, c in enumerate(stx):
            mm(c["keep_lo"], c["H0"])

        rs1 = [[None] * 3 for _ in range(2)]
        for k in (1, 2, 3, 0):
            for s, c in enumerate(stx):
                H0, H1 = c["H0"], c["H1"]
                j = lax.rem(c["g1"] + k, 4)
                off = j * H1
                row = c["keep_lo"] + off
                for r in (1, 2, 3):
                    rs0[s][(r - 1) * 4 + k].wait()
                out_ref[pl.ds(row, H1), :] += (
                    comm_ref[pl.ds(c["comm0"] + off, H1), :].astype(
                        jnp.float32)
                    + comm_ref[pl.ds(c["comm0"] + H0 + off, H1), :].astype(
                        jnp.float32)
                    + comm_ref[
                        pl.ds(c["comm0"] + 2 * H0 + off, H1), :
                    ].astype(jnp.float32)
                )
                if k == 0:
                    continue
                r = 4 - k
                sem = c["sem1"] + r - 1
                st_off = c["stage1"] + (k - 1) * H1
                stage_ref[pl.ds(st_off, H1), :] = out_ref[
                    pl.ds(row, H1), :
                ].astype(jnp.bfloat16)
                rdma = pltpu.make_async_remote_copy(
                    src_ref=stage_ref.at[pl.ds(st_off, H1), :],
                    dst_ref=comm_ref.at[
                        pl.ds(c["comm1"] + (r - 1) * H1, H1), :
                    ],
                    send_sem=send_sems.at[sem],
                    recv_sem=recv_sems.at[sem],
                    device_id=(peer_at(c["units"][1], c["g1"],
                                       c["gbase1"], k),),
                    device_id_type=pl.DeviceIdType.MESH,
                )
                rdma.start()
                rs1[s][k - 1] = rdma

        def ag_send(c, src_lo, H, level, sem_base, d):
            u = c["units"][level]
            g = c["g0"] if level == 0 else c["g1"]
            gb = c["gbase0"] if level == 0 else c["gbase1"]
            r = 4 - d
            rdma = pltpu.make_async_remote_copy(
                src_ref=ag_ref.at[pl.ds(src_lo, H), :],
                dst_ref=ag_ref.at[pl.ds(src_lo, H), :],
                send_sem=send_sems.at[sem_base + r - 1],
                recv_sem=recv_sems.at[sem_base + r - 1],
                device_id=(peer_at(u, g, gb, d),),
                device_id_type=pl.DeviceIdType.MESH,
            )
            rdma.start()
            return rdma

        ag_ctx = [None, None]
        waiters = [[], []]
        for s, c in enumerate(stx):
            H1 = c["H1"]
            chunk_lo = c["keep_lo"] + c["g1"] * H1
            for rdma in rs1[s]:
                rdma.wait()
            out_ref[pl.ds(chunk_lo, H1), :] += (
                comm_ref[pl.ds(c["comm1"], H1), :].astype(jnp.float32)
                + comm_ref[pl.ds(c["comm1"] + H1, H1), :].astype(
                    jnp.float32)
                + comm_ref[pl.ds(c["comm1"] + 2 * H1, H1), :].astype(
                    jnp.float32)
            )
            ag_ref[pl.ds(chunk_lo, H1), :] = out_ref[
                pl.ds(chunk_lo, H1), :
            ].astype(jnp.bfloat16)
            sem_base = ctr["sem"]
            ctr["sem"] += 3
            a2a = [ag_send(c, chunk_lo, H1, 1, sem_base, d)
                   for d in (1, 2, 3)]
            sem_base = ctr["sem"]
            ctr["sem"] += 3
            waiters[s].extend(
                ag_send(c, chunk_lo, H1, 0, sem_base, d)
                for d in (1, 2, 3)
            )
            ag_ctx[s] = (a2a, c["keep_lo"])

        for d in (1, 2, 3):
            for s in (1, 0):
                c = stx[s]
                a2a, block2_lo = ag_ctx[s]
                a2a[d - 1].wait()
                piece_lo = block2_lo + lax.rem(c["g1"] + 4 - d, 4) * c["H1"]
                sem_base = ctr["sem"]
                ctr["sem"] += 3
                waiters[s].extend(
                    ag_send(c, piece_lo, c["H1"], 0, sem_base, dd)
                    for dd in (1, 2, 3)
                )

        for s in (1, 0):
            for rdma in waiters[s]:
                rdma.wait()
            base, R, _ = STREAMS[s]
            out_ref[pl.ds(base, R), :] = ag_ref[
                pl.ds(base, R), :
            ].astype(jnp.float32)

    return pl.pallas_call(
        body,
        out_shape=jax.ShapeDtypeStruct((M, N), jnp.float32),
        in_specs=[
            pl.BlockSpec(memory_space=pltpu.VMEM),
            pl.BlockSpec(memory_space=pltpu.VMEM),
        ],
        out_specs=pl.BlockSpec(memory_space=pltpu.VMEM),
        scratch_shapes=[
            pltpu.VMEM((RS_ROWS, N), jnp.bfloat16),
            pltpu.VMEM((RS_ROWS, N), jnp.bfloat16),
            pltpu.VMEM((M, N), jnp.bfloat16),
            pltpu.SemaphoreType.DMA((N_EXCH,)),
            pltpu.SemaphoreType.DMA((N_EXCH,)),
        ],
        compiler_params=pltpu.CompilerParams(collective_id=0),
    )(t, W)
